# Initial kernel scaffold; baseline (speedup 1.0000x reference)
import jax
import jax.numpy as jnp
from jax.experimental import pallas as pl


def kernel(tokens, table, meta_reps):
    raise NotImplementedError("write your pallas kernel here")


# speedup vs baseline: 2.1198x; 2.1198x over previous
import jax
import jax.numpy as jnp
from jax import lax
from jax.experimental import pallas as pl
from jax.experimental.pallas import tpu as pltpu
from jax.experimental.pallas import tpu_sc as plsc

DIM = 1024
N_MARKERS = 3
NW = 32
CH = 32
NCH = 8
TPW = CH * NCH
NG = TPW // 16


def _body(tok_hbm, table_hbm, meta_hbm, out_hbm,
          tok_v, meta_v, buf0, buf1, gsem0, gsem1, osem0, osem1, msem):
    c = lax.axis_index("c")
    s = lax.axis_index("s")
    wid = s * 2 + c
    base = wid * TPW
    ncols = tok_hbm.shape[1]
    row = (wid * TPW) // ncols
    col = pl.multiple_of((wid * TPW) % ncols, TPW)

    pltpu.sync_copy(tok_hbm.at[row, pl.ds(col, TPW)], tok_v)
    meta_cp = pltpu.async_copy(meta_hbm, meta_v, msem)

    bufs = (buf0, buf1)
    gsems = (gsem0, gsem1)
    osems = (osem0, osem1)

    def gather(j):
        return pltpu.async_copy(
            table_hbm.at[tok_v.at[pl.ds(j * CH, CH)]], bufs[j % 2],
            gsems[j % 2])

    gh = [None] * NCH
    oh = [None, None]
    gh[0] = gather(0)
    for j in range(NCH):
        b = j % 2
        if j + 1 < NCH:
            nb = (j + 1) % 2
            if oh[nb] is not None:
                oh[nb].wait()
            gh[j + 1] = gather(j + 1)
        gh[j].wait()
        oh[b] = pltpu.async_copy(
            bufs[b], out_hbm.at[pl.ds(base + j * CH, CH)], osems[b])
    for h in oh:
        h.wait()

    meta_cp.wait()
    lanes = lax.iota(jnp.int32, 16)

    def lane_min(v):
        for sh in (1, 2, 4, 8):
            v = jnp.minimum(v, v.at[lanes ^ sh].get(mode="promise_in_bounds"))
        return v

    def min_body(g, m):
        off = pl.multiple_of(g * 16, 16)
        return jnp.minimum(m, tok_v[pl.ds(off, 16)])

    gm = lane_min(lax.fori_loop(1, NG, min_body, tok_v[pl.ds(0, 16)]))

    @pl.when(gm[0] < N_MARKERS)
    def _():
        def group_body(g, carry):
            off = pl.multiple_of(g * 16, 16)
            t16 = tok_v[pl.ds(off, 16)]

            @pl.when(lane_min(t16)[0] < N_MARKERS)
            def _():
                def fix(l, c2):
                    sel = (lanes + l) & 15
                    t = t16.at[sel].get(mode="promise_in_bounds")[0]

                    @pl.when(t < N_MARKERS)
                    def __():
                        pltpu.sync_copy(
                            meta_v.at[t],
                            out_hbm.at[base + g * 16 + l])
                    return c2

                lax.fori_loop(0, 16, fix, 0)
            return carry

        lax.fori_loop(0, NG, group_body, 0)


@jax.jit
def _run(tokens, table, meta_reps):
    return pl.kernel(
        _body,
        out_type=jax.ShapeDtypeStruct((NW * TPW, DIM), jnp.float32),
        mesh=plsc.VectorSubcoreMesh(core_axis_name="c", subcore_axis_name="s"),
        scratch_types=[
            pltpu.VMEM((TPW,), jnp.int32),
            pltpu.VMEM((N_MARKERS, DIM), jnp.float32),
            pltpu.VMEM((CH, DIM), jnp.float32),
            pltpu.VMEM((CH, DIM), jnp.float32),
            pltpu.SemaphoreType.DMA,
            pltpu.SemaphoreType.DMA,
            pltpu.SemaphoreType.DMA,
            pltpu.SemaphoreType.DMA,
            pltpu.SemaphoreType.DMA,
        ],
    )(tokens, table, meta_reps)


def kernel(tokens, table, meta_reps):
    b, s = tokens.shape
    out = _run(tokens, table, meta_reps)
    return out.reshape(b, s, DIM)


# speedup vs baseline: 2.1401x; 1.0096x over previous
import jax
import jax.numpy as jnp
from jax import lax
from jax.experimental import pallas as pl
from jax.experimental.pallas import tpu as pltpu
from jax.experimental.pallas import tpu_sc as plsc

DIM = 1024
N_MARKERS = 3
NW = 32
CH = 16
NCH = 16
NB = 4
TPW = CH * NCH
NG = TPW // 16


def _body(tok_hbm, table_hbm, meta_hbm, out_hbm,
          tok_v, meta_v, buf0, buf1, buf2, buf3,
          gsem0, gsem1, gsem2, gsem3, osem0, osem1, osem2, osem3, msem):
    c = lax.axis_index("c")
    s = lax.axis_index("s")
    wid = s * 2 + c
    base = wid * TPW
    ncols = tok_hbm.shape[1]
    row = (wid * TPW) // ncols
    col = pl.multiple_of((wid * TPW) % ncols, TPW)

    pltpu.sync_copy(tok_hbm.at[row, pl.ds(col, TPW)], tok_v)
    meta_cp = pltpu.async_copy(meta_hbm, meta_v, msem)

    bufs = (buf0, buf1, buf2, buf3)
    gsems = (gsem0, gsem1, gsem2, gsem3)
    osems = (osem0, osem1, osem2, osem3)

    def gather(j):
        return pltpu.async_copy(
            table_hbm.at[tok_v.at[pl.ds(j * CH, CH)]], bufs[j % NB],
            gsems[j % NB])

    gh = [None] * NCH
    oh = [None] * NB
    for j in range(NB - 1):
        gh[j] = gather(j)
    for j in range(NCH):
        b = j % NB
        jn = j + NB - 1
        if jn < NCH:
            nb = jn % NB
            if oh[nb] is not None:
                oh[nb].wait()
            gh[jn] = gather(jn)
        gh[j].wait()
        oh[b] = pltpu.async_copy(
            bufs[b], out_hbm.at[pl.ds(base + j * CH, CH)], osems[b])
    for h in oh:
        h.wait()

    meta_cp.wait()
    lanes = lax.iota(jnp.int32, 16)

    def lane_min(v):
        for sh in (1, 2, 4, 8):
            v = jnp.minimum(v, v.at[lanes ^ sh].get(mode="promise_in_bounds"))
        return v

    def min_body(g, m):
        off = pl.multiple_of(g * 16, 16)
        return jnp.minimum(m, tok_v[pl.ds(off, 16)])

    gm = lane_min(lax.fori_loop(1, NG, min_body, tok_v[pl.ds(0, 16)]))

    @pl.when(gm[0] < N_MARKERS)
    def _():
        def group_body(g, carry):
            off = pl.multiple_of(g * 16, 16)
            t16 = tok_v[pl.ds(off, 16)]

            @pl.when(lane_min(t16)[0] < N_MARKERS)
            def _():
                def fix(l, c2):
                    sel = (lanes + l) & 15
                    t = t16.at[sel].get(mode="promise_in_bounds")[0]

                    @pl.when(t < N_MARKERS)
                    def __():
                        pltpu.sync_copy(
                            meta_v.at[t],
                            out_hbm.at[base + g * 16 + l])
                    return c2

                lax.fori_loop(0, 16, fix, 0)
            return carry

        lax.fori_loop(0, NG, group_body, 0)


@jax.jit
def _run(tokens, table, meta_reps):
    return pl.kernel(
        _body,
        out_type=jax.ShapeDtypeStruct((NW * TPW, DIM), jnp.float32),
        mesh=plsc.VectorSubcoreMesh(core_axis_name="c", subcore_axis_name="s"),
        scratch_types=[
            pltpu.VMEM((TPW,), jnp.int32),
            pltpu.VMEM((N_MARKERS, DIM), jnp.float32),
            pltpu.VMEM((CH, DIM), jnp.float32),
            pltpu.VMEM((CH, DIM), jnp.float32),
            pltpu.VMEM((CH, DIM), jnp.float32),
            pltpu.VMEM((CH, DIM), jnp.float32),
            pltpu.SemaphoreType.DMA,
            pltpu.SemaphoreType.DMA,
            pltpu.SemaphoreType.DMA,
            pltpu.SemaphoreType.DMA,
            pltpu.SemaphoreType.DMA,
            pltpu.SemaphoreType.DMA,
            pltpu.SemaphoreType.DMA,
            pltpu.SemaphoreType.DMA,
            pltpu.SemaphoreType.DMA,
        ],
    )(tokens, table, meta_reps)


def kernel(tokens, table, meta_reps):
    b, s = tokens.shape
    out = _run(tokens, table, meta_reps)
    return out.reshape(b, s, DIM)


# speedup vs baseline: 2.1718x; 1.0148x over previous
import jax
import jax.numpy as jnp
from jax import lax
from jax.experimental import pallas as pl
from jax.experimental.pallas import tpu as pltpu
from jax.experimental.pallas import tpu_sc as plsc

DIM = 1024
N_MARKERS = 3
NW = 32
CH = 32
NCH = 8
NB = 3
TPW = CH * NCH
NG = TPW // 16


def _body(tok_hbm, table_hbm, meta_hbm, out_hbm,
          tok_v, meta_v, buf0, buf1, buf2,
          gsem0, gsem1, gsem2, osem0, osem1, osem2, msem):
    c = lax.axis_index("c")
    s = lax.axis_index("s")
    wid = s * 2 + c
    base = wid * TPW
    ncols = tok_hbm.shape[1]
    row = (wid * TPW) // ncols
    col = pl.multiple_of((wid * TPW) % ncols, TPW)

    pltpu.sync_copy(tok_hbm.at[row, pl.ds(col, TPW)], tok_v)
    meta_cp = pltpu.async_copy(meta_hbm, meta_v, msem)

    lanes = lax.iota(jnp.int32, 16)

    def lane_min(v):
        for sh in (1, 2, 4, 8):
            v = jnp.minimum(v, v.at[lanes ^ sh].get(mode="promise_in_bounds"))
        return v

    def min_body(g, m):
        off = pl.multiple_of(g * 16, 16)
        return jnp.minimum(m, tok_v[pl.ds(off, 16)])

    gm = lane_min(lax.fori_loop(1, NG, min_body, tok_v[pl.ds(0, 16)]))

    bufs = (buf0, buf1, buf2)
    gsems = (gsem0, gsem1, gsem2)
    osems = (osem0, osem1, osem2)

    def gather(j):
        return pltpu.async_copy(
            table_hbm.at[tok_v.at[pl.ds(j * CH, CH)]], bufs[j % NB],
            gsems[j % NB])

    gh = [None] * NCH
    oh = [None] * NB
    for j in range(NB - 1):
        gh[j] = gather(j)
    for j in range(NCH):
        b = j % NB
        jn = j + NB - 1
        if jn < NCH:
            nb = jn % NB
            if oh[nb] is not None:
                oh[nb].wait()
            gh[jn] = gather(jn)
        gh[j].wait()
        oh[b] = pltpu.async_copy(
            bufs[b], out_hbm.at[pl.ds(base + j * CH, CH)], osems[b])
    for h in oh:
        h.wait()

    meta_cp.wait()

    @pl.when(gm[0] < N_MARKERS)
    def _():
        def group_body(g, carry):
            off = pl.multiple_of(g * 16, 16)
            t16 = tok_v[pl.ds(off, 16)]

            @pl.when(lane_min(t16)[0] < N_MARKERS)
            def _():
                def fix(l, c2):
                    sel = (lanes + l) & 15
                    t = t16.at[sel].get(mode="promise_in_bounds")[0]

                    @pl.when(t < N_MARKERS)
                    def __():
                        pltpu.sync_copy(
                            meta_v.at[t],
                            out_hbm.at[base + g * 16 + l])
                    return c2

                lax.fori_loop(0, 16, fix, 0)
            return carry

        lax.fori_loop(0, NG, group_body, 0)


@jax.jit
def _run(tokens, table, meta_reps):
    return pl.kernel(
        _body,
        out_type=jax.ShapeDtypeStruct((NW * TPW, DIM), jnp.float32),
        mesh=plsc.VectorSubcoreMesh(core_axis_name="c", subcore_axis_name="s"),
        scratch_types=[
            pltpu.VMEM((TPW,), jnp.int32),
            pltpu.VMEM((N_MARKERS, DIM), jnp.float32),
            pltpu.VMEM((CH, DIM), jnp.float32),
            pltpu.VMEM((CH, DIM), jnp.float32),
            pltpu.VMEM((CH, DIM), jnp.float32),
            pltpu.SemaphoreType.DMA,
            pltpu.SemaphoreType.DMA,
            pltpu.SemaphoreType.DMA,
            pltpu.SemaphoreType.DMA,
            pltpu.SemaphoreType.DMA,
            pltpu.SemaphoreType.DMA,
            pltpu.SemaphoreType.DMA,
        ],
    )(tokens, table, meta_reps)


def kernel(tokens, table, meta_reps):
    b, s = tokens.shape
    out = _run(tokens, table, meta_reps)
    return out.reshape(b, s, DIM)
